# baseline (speedup 1.0000x reference)
import functools

import jax
import jax.numpy as jnp
from jax import lax
from jax.experimental import pallas as pl
from jax.experimental.pallas import tpu as pltpu
from jax.experimental.pallas import tpu_sc as plsc

V = 10000
H = 64
E = 320000
NC = 2
NS = 16
NW = NC * NS
EPW = E // NW
CH = 80
NCHUNK = EPW // CH
NB = 5
RPS = V // NS


def _mm(x, w):
    return jnp.dot(x, w, preferred_element_type=jnp.float32)


def _t0_body(x_ref, wi_ref, wm_ref, h_ref, a_ref):
    h = jnp.tanh(_mm(x_ref[...], wi_ref[...]))
    h_ref[...] = h
    a_ref[...] = jnp.maximum(_mm(h, wm_ref[...]), 0.0)


def _t_dense_body(p_ref, wd_ref, wm_ref, a_ref):
    s = p_ref[0] + p_ref[1]
    c = jnp.tanh(_mm(s, wd_ref[...]))
    a_ref[...] = jnp.maximum(_mm(c, wm_ref[...]), 0.0)


def _t_res_body(p_ref, h_ref, wm_ref, a_ref):
    m = (p_ref[0] + p_ref[1] + h_ref[...]) * 0.5
    a_ref[...] = jnp.maximum(_mm(m, wm_ref[...]), 0.0)


def _t_sum_body(p_ref, o_ref):
    o_ref[...] = p_ref[0] + p_ref[1]


_F = jax.ShapeDtypeStruct


def _tc0(x, wi, wm):
    return pl.pallas_call(
        _t0_body,
        out_shape=(_F((V, H), jnp.float32), _F((V, H), jnp.float32)),
    )(x, wi, wm)


def _tc_dense(p, wd, wm):
    return pl.pallas_call(
        _t_dense_body, out_shape=_F((V, H), jnp.float32))(p, wd, wm)


def _tc_res(p, h, wm):
    return pl.pallas_call(
        _t_res_body, out_shape=_F((V, H), jnp.float32))(p, h, wm)


def _tc_sum(p):
    return pl.pallas_call(_t_sum_body, out_shape=_F((V, H), jnp.float32))(p)


_mesh = plsc.VectorSubcoreMesh(core_axis_name="c", subcore_axis_name="s")


@functools.partial(
    pl.kernel,
    out_type=_F((NC, NS, RPS, H), jnp.float32),
    mesh=_mesh,
    scratch_types=[
        pltpu.VMEM((NCHUNK, CH), jnp.int32),
        pltpu.VMEM((NCHUNK, CH), jnp.int32),
        pltpu.VMEM((NB, CH, H), jnp.float32),
        pltpu.VMEM_SHARED((V, H), jnp.float32),
        pltpu.VMEM_SHARED((V, H), jnp.float32),
        pltpu.SemaphoreType.DMA((NB,)),
        pltpu.SemaphoreType.DMA((NB,)),
    ],
    compiler_params=pltpu.CompilerParams(use_tc_tiling_on_sc=False),
)
def _sc_edge_pass(a_hbm, src_hbm, dst_hbm, z_hbm, out_hbm,
                  srcv, dstv, rows, acc, table, gsem, ssem):
    c = lax.axis_index("c")
    s = lax.axis_index("s")
    wid = c * NS + s

    pltpu.sync_copy(z_hbm.at[s], acc.at[pl.ds(s * RPS, RPS)])
    pltpu.sync_copy(a_hbm.at[pl.ds(s * RPS, RPS)],
                    table.at[pl.ds(s * RPS, RPS)])
    pltpu.sync_copy(src_hbm.at[wid], srcv)
    pltpu.sync_copy(dst_hbm.at[wid], dstv)
    plsc.subcore_barrier()

    def _start_gather(j, b):
        pltpu.async_copy(table.at[srcv.at[j]], rows.at[b], gsem.at[b])

    def _wait_gather(b):
        pltpu.make_async_copy(
            table.at[srcv.at[0]], rows.at[b], gsem.at[b]).wait()

    def _start_scatter(j, b):
        pltpu.async_copy(rows.at[b], acc.at[dstv.at[j]], ssem.at[b], add=True)

    def _wait_scatter(b):
        pltpu.make_async_copy(
            rows.at[0], acc.at[dstv.at[0]], ssem.at[b]).wait()

    for b in range(NB):
        _start_gather(b, b)

    @pl.loop(0, NCHUNK, step=NB)
    def _(j):
        for b in range(NB):
            _wait_gather(b)
            _start_scatter(j + b, b)
        for b in range(NB):
            _wait_scatter(b)

            @pl.when(j + b + NB < NCHUNK)
            def _():
                _start_gather(j + b + NB, b)

    plsc.subcore_barrier()
    pltpu.sync_copy(acc.at[pl.ds(s * RPS, RPS)], out_hbm.at[c, s])


def kernel(node_features, adjacency_list_0, node_to_graph_map, num_graphs,
           W_init, W_mp0, W_mp1, W_mp2, W_mp3, W_dense0, W_dense2):
    src3 = adjacency_list_0[:, 0].reshape(NW, NCHUNK, CH)
    dst3 = adjacency_list_0[:, 1].reshape(NW, NCHUNK, CH)
    zeros = jnp.zeros((NS, RPS, H), jnp.float32)

    def edge_pass(a):
        return _sc_edge_pass(a, src3, dst3, zeros).reshape(NC, V, H)

    h0, a0 = _tc0(node_features, W_init, W_mp0)
    p0 = edge_pass(a0)
    a1 = _tc_dense(p0, W_dense0, W_mp1)
    p1 = edge_pass(a1)
    a2 = _tc_res(p1, h0, W_mp2)
    p2 = edge_pass(a2)
    a3 = _tc_dense(p2, W_dense2, W_mp3)
    p3 = edge_pass(a3)
    return _tc_sum(p3)


# speedup vs baseline: 1.4578x; 1.4578x over previous
import functools

import jax
import jax.numpy as jnp
from jax import lax
from jax.experimental import pallas as pl
from jax.experimental.pallas import tpu as pltpu
from jax.experimental.pallas import tpu_sc as plsc

V = 10000
H = 64
E = 320000
NC = 2
NS = 16
NW = NC * NS
EPW = E // NW
CH = 80
NCHUNK = EPW // CH
NB = 5
RPS = V // NS


def _mm(x, w):
    return jnp.dot(x, w, preferred_element_type=jnp.float32)


def _t0_body(x_ref, wi_ref, wm_ref, h_ref, a_ref):
    h = jnp.tanh(_mm(x_ref[...], wi_ref[...]))
    h_ref[...] = h
    a_ref[...] = jnp.maximum(_mm(h, wm_ref[...]), 0.0)


def _t_dense_body(p_ref, wd_ref, wm_ref, a_ref):
    s = p_ref[0] + p_ref[1]
    c = jnp.tanh(_mm(s, wd_ref[...]))
    a_ref[...] = jnp.maximum(_mm(c, wm_ref[...]), 0.0)


def _t_res_body(p_ref, h_ref, wm_ref, a_ref):
    m = (p_ref[0] + p_ref[1] + h_ref[...]) * 0.5
    a_ref[...] = jnp.maximum(_mm(m, wm_ref[...]), 0.0)


def _t_sum_body(p_ref, o_ref):
    o_ref[...] = p_ref[0] + p_ref[1]


_F = jax.ShapeDtypeStruct


def _tc0(x, wi, wm):
    return pl.pallas_call(
        _t0_body,
        out_shape=(_F((V, H), jnp.float32), _F((V, H), jnp.float32)),
    )(x, wi, wm)


def _tc_dense(p, wd, wm):
    return pl.pallas_call(
        _t_dense_body, out_shape=_F((V, H), jnp.float32))(p, wd, wm)


def _tc_res(p, h, wm):
    return pl.pallas_call(
        _t_res_body, out_shape=_F((V, H), jnp.float32))(p, h, wm)


def _tc_sum(p):
    return pl.pallas_call(_t_sum_body, out_shape=_F((V, H), jnp.float32))(p)


_mesh = plsc.VectorSubcoreMesh(core_axis_name="c", subcore_axis_name="s")


@functools.partial(
    pl.kernel,
    out_type=_F((NC, NS, RPS, H), jnp.float32),
    mesh=_mesh,
    scratch_types=[
        pltpu.VMEM((NCHUNK, CH), jnp.int32),
        pltpu.VMEM((NCHUNK, CH), jnp.int32),
        pltpu.VMEM((NB, CH, H), jnp.float32),
        pltpu.VMEM_SHARED((V, H), jnp.float32),
        pltpu.SemaphoreType.DMA((NB,)),
    ],
    compiler_params=pltpu.CompilerParams(use_tc_tiling_on_sc=False),
)
def _sc_edge_pass(a_hbm, src_hbm, dst_hbm, z_hbm, out_hbm,
                  srcv, dstv, rows, acc, gsem):
    c = lax.axis_index("c")
    s = lax.axis_index("s")
    wid = c * NS + s

    pltpu.sync_copy(z_hbm.at[s], acc.at[pl.ds(s * RPS, RPS)])
    pltpu.sync_copy(src_hbm.at[wid], srcv)
    pltpu.sync_copy(dst_hbm.at[wid], dstv)
    plsc.subcore_barrier()

    def _start_gather(j, b):
        pltpu.async_copy(a_hbm.at[srcv.at[j]], rows.at[b], gsem.at[b])

    def _wait_gather(b):
        pltpu.make_async_copy(
            a_hbm.at[srcv.at[0]], rows.at[b], gsem.at[b]).wait()

    def _scatter(j, b):
        pltpu.sync_copy(rows.at[b], acc.at[dstv.at[j]], add=True)

    for b in range(NB):
        _start_gather(b, b)

    @pl.loop(0, NCHUNK, step=NB)
    def _(j):
        for b in range(NB):
            _wait_gather(b)
            _scatter(j + b, b)

            @pl.when(j + b + NB < NCHUNK)
            def _():
                _start_gather(j + b + NB, b)

    plsc.subcore_barrier()
    pltpu.sync_copy(acc.at[pl.ds(s * RPS, RPS)], out_hbm.at[c, s])


def kernel(node_features, adjacency_list_0, node_to_graph_map, num_graphs,
           W_init, W_mp0, W_mp1, W_mp2, W_mp3, W_dense0, W_dense2):
    src3 = adjacency_list_0[:, 0].reshape(NW, NCHUNK, CH)
    dst3 = adjacency_list_0[:, 1].reshape(NW, NCHUNK, CH)
    zeros = jnp.zeros((NS, RPS, H), jnp.float32)

    def edge_pass(a):
        return _sc_edge_pass(a, src3, dst3, zeros).reshape(NC, V, H)

    h0, a0 = _tc0(node_features, W_init, W_mp0)
    p0 = edge_pass(a0)
    a1 = _tc_dense(p0, W_dense0, W_mp1)
    p1 = edge_pass(a1)
    a2 = _tc_res(p1, h0, W_mp2)
    p2 = edge_pass(a2)
    a3 = _tc_dense(p2, W_dense2, W_mp3)
    p3 = edge_pass(a3)
    return _tc_sum(p3)
